# baseline (speedup 1.0000x reference)
import functools

import jax
import jax.numpy as jnp
from jax import lax
from jax.experimental import pallas as pl
from jax.experimental.pallas import tpu as pltpu
from jax.experimental.pallas import tpu_sc as plsc

_B = 4096
_H = 200
_D = 64
_C = 128

_NC = 2
_NS = 16
_NW = _NC * _NS
_BPW = _B // _NW

_G1 = 128
_G2 = _H - _G1


def _sc_pool_body(x_hbm, table_hbm, out_hbm, idx_v, gbuf, out_v,
                  sa0, sb0, sa1, sb1):
    wid = lax.axis_index("c") * _NS + lax.axis_index("s")
    base = wid * _BPW

    pltpu.sync_copy(x_hbm.at[pl.ds(base, _BPW)], idx_v)

    sems = ((sa0, sb0), (sa1, sb1))

    def issue(b, s):
        pltpu.async_copy(table_hbm.at[idx_v.at[b, pl.ds(0, _G1)]],
                         gbuf.at[s, pl.ds(0, _G1)], sems[s][0])
        pltpu.async_copy(table_hbm.at[idx_v.at[b, pl.ds(_G1, _G2)]],
                         gbuf.at[s, pl.ds(_G1, _G2)], sems[s][1])

    def wait_slot(s):
        pltpu.make_async_copy(table_hbm.at[pl.ds(0, _G1)],
                              gbuf.at[s, pl.ds(0, _G1)], sems[s][0]).wait()
        pltpu.make_async_copy(table_hbm.at[pl.ds(0, _G2)],
                              gbuf.at[s, pl.ds(_G1, _G2)], sems[s][1]).wait()

    def accum(b, s):
        unroll = 8

        def rbody(i, accs):
            accs = list(accs)
            for u in range(unroll):
                r = i * unroll + u
                for d in range(_D // 16):
                    accs[d] = accs[d] + gbuf[s, r, pl.ds(d * 16, 16)]
            return tuple(accs)

        zero = jnp.zeros((16,), jnp.float32)
        accs = lax.fori_loop(0, _H // unroll, rbody, (zero,) * (_D // 16))
        for d in range(_D // 16):
            out_v[b, pl.ds(d * 16, 16)] = accs[d]

    issue(0, 0)
    issue(1, 1)

    def body(b2, carry):
        for s in range(2):
            b = b2 * 2 + s
            wait_slot(s)
            accum(b, s)

            @pl.when(b + 2 < _BPW)
            def _():
                issue(b + 2, s)
        return carry

    lax.fori_loop(0, _BPW // 2, body, 0)

    pltpu.sync_copy(out_v, out_hbm.at[pl.ds(base, _BPW)])


_sc_pool = functools.partial(
    pl.kernel,
    out_type=jax.ShapeDtypeStruct((_B, _D), jnp.float32),
    mesh=plsc.VectorSubcoreMesh(core_axis_name="c", subcore_axis_name="s"),
    compiler_params=pltpu.CompilerParams(use_tc_tiling_on_sc=False),
    scratch_types=[
        pltpu.VMEM((_BPW, _H), jnp.int32),
        pltpu.VMEM((2, _H, _D), jnp.float32),
        pltpu.VMEM((_BPW, _D), jnp.float32),
        pltpu.SemaphoreType.DMA,
        pltpu.SemaphoreType.DMA,
        pltpu.SemaphoreType.DMA,
        pltpu.SemaphoreType.DMA,
    ],
)(_sc_pool_body)


def _fc_body(p_ref, len_ref, w_ref, b_ref, o_ref):
    acc = jnp.dot(p_ref[...], w_ref[...], preferred_element_type=jnp.float32)
    o_ref[...] = acc / len_ref[...] + b_ref[...]


_BB = 256


def _fc(pooled, len_col, w_t, bias_row):
    return pl.pallas_call(
        _fc_body,
        out_shape=jax.ShapeDtypeStruct((_B, _C), jnp.float32),
        grid=(_B // _BB,),
        in_specs=[
            pl.BlockSpec((_BB, _D), lambda i: (i, 0)),
            pl.BlockSpec((_BB, 1), lambda i: (i, 0)),
            pl.BlockSpec((_D, _C), lambda i: (0, 0)),
            pl.BlockSpec((1, _C), lambda i: (0, 0)),
        ],
        out_specs=pl.BlockSpec((_BB, _C), lambda i: (i, 0)),
    )(pooled, len_col, w_t, bias_row)


def kernel(x, x_len, table, fc_w, fc_b):
    x32 = x.astype(jnp.int32)
    pooled = _sc_pool(x32, table)
    len_col = x_len.astype(jnp.float32).reshape(_B, 1)
    return _fc(pooled, len_col, fc_w.T, fc_b.reshape(1, _C))
